# baseline (speedup 1.0000x reference)
import functools

import jax
import jax.numpy as jnp
from jax import lax
from jax.experimental import pallas as pl
from jax.experimental.pallas import tpu as pltpu
from jax.experimental.pallas import tpu_sc as plsc

B, C, D = 16384, 100, 64
ROW = C * D
NC, NS, L = 2, 16, 16
NW = NC * NS
BPW = B // NW
CHUNK = 8
NCHUNK = BPW // CHUNK
NJ = ROW // L

_mesh = plsc.VectorSubcoreMesh(core_axis_name="c", subcore_axis_name="s")


@functools.partial(
    pl.kernel,
    mesh=_mesh,
    out_type=jax.ShapeDtypeStruct((B, ROW), jnp.float32),
    scratch_types=[
        pltpu.VMEM((ROW,), jnp.float32),
        pltpu.VMEM((CHUNK, ROW), jnp.float32),
    ],
)
def _col_add(x_hbm, t_hbm, o_hbm, tbuf, buf):
    wid = lax.axis_index("s") * NC + lax.axis_index("c")
    base = wid * BPW
    pltpu.sync_copy(t_hbm, tbuf)

    def chunk_body(i, carry):
        r0 = base + i * CHUNK
        pltpu.sync_copy(x_hbm.at[pl.ds(r0, CHUNK)], buf)

        def j_body(j, c2):
            sl = pl.ds(j * L, L)
            t = tbuf[sl]
            for r in range(CHUNK):
                buf[r, sl] += t
            return c2

        lax.fori_loop(0, NJ, j_body, 0)
        pltpu.sync_copy(buf, o_hbm.at[pl.ds(r0, CHUNK)])
        return carry

    lax.fori_loop(0, NCHUNK, chunk_body, 0)


def kernel(inputs, table):
    out = _col_add(inputs.reshape(B, ROW), table.reshape(ROW))
    return out.reshape(B, C, D)
